# baseline (speedup 1.0000x reference)
import math

import jax
import jax.numpy as jnp
from jax.experimental import pallas as pl
from jax.experimental.pallas import tpu as pltpu

INV_SQRT2 = 1.0 / math.sqrt(2.0)
NEG_SLOPE = 0.2

H = 32
W = 32
HW = H * W
PLANES = 32
TAPS = 9
IMGS_PER_STEP = 2


def _lrelu(v):
    return jnp.where(v >= 0.0, v, NEG_SLOPE * v)


def _combine_taps(parts):
    q = jax.lax.broadcasted_iota(jnp.int32, (PLANES, HW), 1)
    hh = q // W
    ww = q % W
    acc = None
    for t in range(TAPS):
        dy = t // 3 - 1
        dx = t % 3 - 1
        z = parts[t * PLANES:(t + 1) * PLANES, :]
        off = dy * W + dx
        if off != 0:
            z = jnp.roll(z, -off, axis=1)
        cond = None
        for c in ((hh >= 1) if dy == -1 else None,
                  (hh <= H - 2) if dy == 1 else None,
                  (ww >= 1) if dx == -1 else None,
                  (ww <= W - 2) if dx == 1 else None):
            if c is not None:
                cond = c if cond is None else (cond & c)
        if cond is not None:
            z = jnp.where(cond, z, 0.0)
        acc = z if acc is None else acc + z
    return acc


def _block_kernel(x_ref, wbig_ref, wc1_ref, w2_ref, wc2_ref, bias_ref,
                  coords_ref, o_ref):
    coords = coords_ref[...]
    b1 = bias_ref[:, 0:1]
    b2 = bias_ref[:, 1:2]
    bp = bias_ref[:, 2:3]

    for img in range(IMGS_PER_STEP):
        x = x_ref[img].astype(jnp.bfloat16).reshape(-1, HW)

        a = jnp.dot(wbig_ref[...], x, preferred_element_type=jnp.float32)
        a = a + jnp.dot(wc1_ref[...], coords,
                        preferred_element_type=jnp.float32)

        y1 = _lrelu(_combine_taps(a[:TAPS * PLANES]) + b1)

        b = jnp.dot(w2_ref[...], y1.astype(jnp.bfloat16),
                    preferred_element_type=jnp.float32)
        b = b + jnp.dot(wc2_ref[...], coords,
                        preferred_element_type=jnp.float32)
        y2 = _lrelu(_combine_taps(b) + b2)

        ident = a[TAPS * PLANES:TAPS * PLANES + PLANES] + bp
        o_ref[img] = ((y2 + ident) * INV_SQRT2).reshape(PLANES, H, W)


def _tap_major(w):
    cout, cin = w.shape[0], w.shape[1]
    return w.transpose(2, 3, 0, 1).reshape(TAPS * cout, cin)


def kernel(w1, b1, w2, b2, wproj, bproj, x):
    B, Cin = x.shape[0], x.shape[1]
    x4 = x.astype(jnp.float32)

    w1f = w1.astype(jnp.float32)
    w2f = w2.astype(jnp.float32)

    w1_main = _tap_major(w1f[:, :Cin])
    w1_coord = _tap_major(w1f[:, Cin:])
    wbig = jnp.concatenate(
        [w1_main, wproj.astype(jnp.float32).reshape(PLANES, Cin)],
        axis=0).astype(jnp.bfloat16)
    wc1 = jnp.pad(w1_coord, ((0, PLANES), (0, 6))).astype(jnp.bfloat16)

    w2_main = _tap_major(w2f[:, :PLANES]).astype(jnp.bfloat16)
    wc2 = jnp.pad(_tap_major(w2f[:, PLANES:]),
                  ((0, 0), (0, 6))).astype(jnp.bfloat16)

    bias = jnp.stack([b1, b2, bproj], axis=1).astype(jnp.float32)
    bias = jnp.pad(bias, ((0, 0), (0, 5)))

    span = jnp.arange(H, dtype=jnp.float32) / (H - 1) * 2.0 - 1.0
    xx = jnp.broadcast_to(span[:, None], (H, W)).reshape(1, HW)
    yy = jnp.broadcast_to(span[None, :], (H, W)).reshape(1, HW)
    coords = jnp.concatenate(
        [xx, yy, jnp.zeros((6, HW), jnp.float32)], axis=0).astype(jnp.bfloat16)

    out = pl.pallas_call(
        _block_kernel,
        grid=(B // IMGS_PER_STEP,),
        out_shape=jax.ShapeDtypeStruct((B, PLANES, H, W), jnp.float32),
        in_specs=[
            pl.BlockSpec((IMGS_PER_STEP, Cin, H, W), lambda i: (i, 0, 0, 0)),
            pl.BlockSpec(wbig.shape, lambda i: (0, 0)),
            pl.BlockSpec(wc1.shape, lambda i: (0, 0)),
            pl.BlockSpec(w2_main.shape, lambda i: (0, 0)),
            pl.BlockSpec(wc2.shape, lambda i: (0, 0)),
            pl.BlockSpec(bias.shape, lambda i: (0, 0)),
            pl.BlockSpec(coords.shape, lambda i: (0, 0)),
        ],
        out_specs=pl.BlockSpec((IMGS_PER_STEP, PLANES, H, W),
                               lambda i: (i, 0, 0, 0)),
        compiler_params=pltpu.CompilerParams(
            dimension_semantics=("parallel",)),
    )(x4, wbig, wc1, w2_main, wc2, bias, coords)

    return out


# speedup vs baseline: 1.9660x; 1.9660x over previous
import math

import jax
import jax.numpy as jnp
from jax.experimental import pallas as pl
from jax.experimental.pallas import tpu as pltpu

INV_SQRT2 = 1.0 / math.sqrt(2.0)
NEG_SLOPE = 0.2

H = 32
W = 32
HW = H * W
PLANES = 32
TAPS = 9
IMGS_PER_STEP = 2


def _lrelu(v):
    return jnp.where(v >= 0.0, v, NEG_SLOPE * v)


def _combine_taps(parts):
    q = jax.lax.broadcasted_iota(jnp.int32, (PLANES, HW), 1)
    hh = q // W
    ww = q % W
    acc = None
    for t in range(TAPS):
        dy = t // 3 - 1
        dx = t % 3 - 1
        z = parts[t * PLANES:(t + 1) * PLANES, :]
        off = dy * W + dx
        if off != 0:
            z = jnp.roll(z, -off, axis=1)
        cond = None
        for c in ((hh >= 1) if dy == -1 else None,
                  (hh <= H - 2) if dy == 1 else None,
                  (ww >= 1) if dx == -1 else None,
                  (ww <= W - 2) if dx == 1 else None):
            if c is not None:
                cond = c if cond is None else (cond & c)
        if cond is not None:
            z = jnp.where(cond, z, 0.0)
        acc = z if acc is None else acc + z
    return acc


def _block_kernel(x_ref, wbig_ref, wc1_ref, w2_ref, wc2_ref, bias_ref,
                  coords_ref, o_ref):
    coords = coords_ref[...]
    b1 = bias_ref[:, 0:1]
    b2 = bias_ref[:, 1:2]
    bp = bias_ref[:, 2:3]

    for img in range(IMGS_PER_STEP):
        x = x_ref[img]

        a = jnp.dot(wbig_ref[...], x, preferred_element_type=jnp.float32)
        a = a + jnp.dot(wc1_ref[...], coords,
                        preferred_element_type=jnp.float32)

        y1 = _lrelu(_combine_taps(a[:TAPS * PLANES]) + b1)

        b = jnp.dot(w2_ref[...], y1.astype(jnp.bfloat16),
                    preferred_element_type=jnp.float32)
        b = b + jnp.dot(wc2_ref[...], coords,
                        preferred_element_type=jnp.float32)
        y2 = _lrelu(_combine_taps(b) + b2)

        ident = a[TAPS * PLANES:TAPS * PLANES + PLANES] + bp
        o_ref[img] = (y2 + ident) * INV_SQRT2


def _tap_major(w):
    cout, cin = w.shape[0], w.shape[1]
    return w.transpose(2, 3, 0, 1).reshape(TAPS * cout, cin)


def kernel(w1, b1, w2, b2, wproj, bproj, x):
    B, Cin = x.shape[0], x.shape[1]
    x3 = x.astype(jnp.bfloat16).reshape(B, Cin, HW)

    w1f = w1.astype(jnp.float32)
    w2f = w2.astype(jnp.float32)

    w1_main = _tap_major(w1f[:, :Cin])
    w1_coord = _tap_major(w1f[:, Cin:])
    wbig = jnp.concatenate(
        [w1_main, wproj.astype(jnp.float32).reshape(PLANES, Cin)],
        axis=0).astype(jnp.bfloat16)
    wc1 = jnp.pad(w1_coord, ((0, PLANES), (0, 6))).astype(jnp.bfloat16)

    w2_main = _tap_major(w2f[:, :PLANES]).astype(jnp.bfloat16)
    wc2 = jnp.pad(_tap_major(w2f[:, PLANES:]),
                  ((0, 0), (0, 6))).astype(jnp.bfloat16)

    bias = jnp.stack([b1, b2, bproj], axis=1).astype(jnp.float32)
    bias = jnp.pad(bias, ((0, 0), (0, 5)))

    span = jnp.arange(H, dtype=jnp.float32) / (H - 1) * 2.0 - 1.0
    xx = jnp.broadcast_to(span[:, None], (H, W)).reshape(1, HW)
    yy = jnp.broadcast_to(span[None, :], (H, W)).reshape(1, HW)
    coords = jnp.concatenate(
        [xx, yy, jnp.zeros((6, HW), jnp.float32)], axis=0).astype(jnp.bfloat16)

    out = pl.pallas_call(
        _block_kernel,
        grid=(B // IMGS_PER_STEP,),
        out_shape=jax.ShapeDtypeStruct((B, PLANES, HW), jnp.float32),
        in_specs=[
            pl.BlockSpec((IMGS_PER_STEP, Cin, HW), lambda i: (i, 0, 0)),
            pl.BlockSpec(wbig.shape, lambda i: (0, 0)),
            pl.BlockSpec(wc1.shape, lambda i: (0, 0)),
            pl.BlockSpec(w2_main.shape, lambda i: (0, 0)),
            pl.BlockSpec(wc2.shape, lambda i: (0, 0)),
            pl.BlockSpec(bias.shape, lambda i: (0, 0)),
            pl.BlockSpec(coords.shape, lambda i: (0, 0)),
        ],
        out_specs=pl.BlockSpec((IMGS_PER_STEP, PLANES, HW),
                               lambda i: (i, 0, 0)),
        compiler_params=pltpu.CompilerParams(
            dimension_semantics=("parallel",)),
    )(x3, wbig, wc1, w2_main, wc2, bias, coords)

    return out.reshape(B, PLANES, H, W)


# speedup vs baseline: 2.2010x; 1.1195x over previous
import math

import jax
import jax.numpy as jnp
from jax.experimental import pallas as pl
from jax.experimental.pallas import tpu as pltpu

INV_SQRT2 = 1.0 / math.sqrt(2.0)
NEG_SLOPE = 0.2

H = 32
W = 32
HW = H * W
PLANES = 32
TAPS = 9
IMGS_PER_STEP = 2
M1 = TAPS * PLANES
MBIG = M1 + PLANES


def _lrelu(v):
    return jnp.where(v >= 0.0, v, NEG_SLOPE * v)


def _combine_taps(parts):
    q = jax.lax.broadcasted_iota(jnp.int32, (PLANES, HW), 1)
    hh = q // W
    ww = q % W
    acc = None
    for t in range(TAPS):
        dy = t // 3 - 1
        dx = t % 3 - 1
        z = parts[t * PLANES:(t + 1) * PLANES, :]
        off = dy * W + dx
        if off != 0:
            z = jnp.roll(z, -off, axis=1)
        cond = None
        for c in ((hh >= 1) if dy == -1 else None,
                  (hh <= H - 2) if dy == 1 else None,
                  (ww >= 1) if dx == -1 else None,
                  (ww <= W - 2) if dx == 1 else None):
            if c is not None:
                cond = c if cond is None else (cond & c)
        if cond is not None:
            z = jnp.where(cond, z, 0.0)
        acc = z if acc is None else acc + z
    return acc


def _block_kernel(x_ref, wmain_ref, waux_ref, coords_ref, o_ref):
    wbig = wmain_ref[:, :256]
    w2 = wmain_ref[:M1, 256:288]
    wc1 = waux_ref[:MBIG].astype(jnp.bfloat16)
    wc2 = waux_ref[MBIG:MBIG + M1].astype(jnp.bfloat16)
    b1 = waux_ref[MBIG + M1:MBIG + M1 + PLANES, 0:1]
    b2 = waux_ref[MBIG + M1:MBIG + M1 + PLANES, 1:2]
    bp = waux_ref[MBIG + M1:MBIG + M1 + PLANES, 2:3]
    coords = coords_ref[...]

    for img in range(IMGS_PER_STEP):
        x = x_ref[img].astype(jnp.bfloat16)

        a = jnp.dot(wbig, x, preferred_element_type=jnp.float32)
        a = a + jnp.dot(wc1, coords, preferred_element_type=jnp.float32)

        y1 = _lrelu(_combine_taps(a[:M1]) + b1)

        b = jnp.dot(w2, y1.astype(jnp.bfloat16),
                    preferred_element_type=jnp.float32)
        b = b + jnp.dot(wc2, coords, preferred_element_type=jnp.float32)
        y2 = _lrelu(_combine_taps(b) + b2)

        ident = a[M1:MBIG] + bp
        o_ref[img] = ((y2 + ident) * INV_SQRT2).reshape(PLANES, H, W)


def _tap_major(w):
    cout, cin = w.shape[0], w.shape[1]
    return w.transpose(2, 3, 0, 1).reshape(TAPS * cout, cin)


def kernel(w1, b1, w2, b2, wproj, bproj, x):
    B, Cin = x.shape[0], x.shape[1]
    x3 = x.astype(jnp.float32).reshape(B, Cin, HW)

    w1f = w1.astype(jnp.float32)
    w2f = w2.astype(jnp.float32)

    wbig = jnp.concatenate(
        [_tap_major(w1f[:, :Cin]), wproj.astype(jnp.float32).reshape(PLANES, Cin)],
        axis=0)
    w2_main = jnp.pad(_tap_major(w2f[:, :PLANES]), ((0, PLANES), (0, 0)))
    wmain = jnp.concatenate([wbig, w2_main], axis=1).astype(jnp.bfloat16)

    wc1 = jnp.pad(_tap_major(w1f[:, Cin:]), ((0, PLANES), (0, 6)))
    wc2 = jnp.pad(_tap_major(w2f[:, PLANES:]), ((0, 0), (0, 6)))
    bias = jnp.pad(jnp.stack([b1, b2, bproj], axis=1), ((0, 0), (0, 5)))
    waux = jnp.concatenate([wc1, wc2, bias], axis=0)

    span = jnp.arange(H, dtype=jnp.float32) / (H - 1) * 2.0 - 1.0
    xx = jnp.broadcast_to(span[:, None], (H, W)).reshape(1, HW)
    yy = jnp.broadcast_to(span[None, :], (H, W)).reshape(1, HW)
    coords = jnp.concatenate(
        [xx, yy, jnp.zeros((6, HW), jnp.float32)], axis=0).astype(jnp.bfloat16)

    out = pl.pallas_call(
        _block_kernel,
        grid=(B // IMGS_PER_STEP,),
        out_shape=jax.ShapeDtypeStruct((B, PLANES, H, W), jnp.float32),
        in_specs=[
            pl.BlockSpec((IMGS_PER_STEP, Cin, HW), lambda i: (i, 0, 0)),
            pl.BlockSpec(wmain.shape, lambda i: (0, 0)),
            pl.BlockSpec(waux.shape, lambda i: (0, 0)),
            pl.BlockSpec(coords.shape, lambda i: (0, 0)),
        ],
        out_specs=pl.BlockSpec((IMGS_PER_STEP, PLANES, H, W),
                               lambda i: (i, 0, 0, 0)),
        compiler_params=pltpu.CompilerParams(
            dimension_semantics=("parallel",)),
    )(x3, wmain, waux, coords)

    return out


# speedup vs baseline: 2.2355x; 1.0157x over previous
import math

import jax
import jax.numpy as jnp
from jax.experimental import pallas as pl
from jax.experimental.pallas import tpu as pltpu

INV_SQRT2 = 1.0 / math.sqrt(2.0)
NEG_SLOPE = 0.2

H = 32
W = 32
HW = H * W
PLANES = 32
TAPS = 9
IMGS_PER_STEP = 2
M1 = TAPS * PLANES


def _lrelu(v):
    return jnp.where(v >= 0.0, v, NEG_SLOPE * v)


def _combine_taps(parts):
    q = jax.lax.broadcasted_iota(jnp.int32, (PLANES, HW), 1)
    hh = q // W
    ww = q % W
    acc = None
    for t in range(TAPS):
        dy = t // 3 - 1
        dx = t % 3 - 1
        z = parts[t * PLANES:(t + 1) * PLANES, :]
        off = dy * W + dx
        if off != 0:
            z = jnp.roll(z, -off, axis=1)
        cond = None
        for c in ((hh >= 1) if dy == -1 else None,
                  (hh <= H - 2) if dy == 1 else None,
                  (ww >= 1) if dx == -1 else None,
                  (ww <= W - 2) if dx == 1 else None):
            if c is not None:
                cond = c if cond is None else (cond & c)
        if cond is not None:
            z = jnp.where(cond, z, 0.0)
        acc = z if acc is None else acc + z
    return acc


def _block_kernel(x_ref, w1t_ref, wp_ref, w2t_ref, bias_ref, coords_ref,
                  o_ref):
    w1m = w1t_ref[:, :256]
    w1c = w1t_ref[:, 256:258]
    w2m = w2t_ref[:, :PLANES]
    w2c = w2t_ref[:, PLANES:PLANES + 2]
    wp = wp_ref[...]
    b1 = bias_ref[:, 0:1]
    b2 = bias_ref[:, 1:2]
    bp = bias_ref[:, 2:3]
    coords = coords_ref[:2]

    for img in range(IMGS_PER_STEP):
        x = x_ref[img].astype(jnp.bfloat16).reshape(-1, HW)

        a = jnp.dot(w1m, x, preferred_element_type=jnp.float32)
        a = a + jnp.dot(w1c, coords, preferred_element_type=jnp.float32)
        y1 = _lrelu(_combine_taps(a) + b1)

        ident = jnp.dot(wp, x, preferred_element_type=jnp.float32) + bp

        b = jnp.dot(w2m, y1.astype(jnp.bfloat16),
                    preferred_element_type=jnp.float32)
        b = b + jnp.dot(w2c, coords, preferred_element_type=jnp.float32)
        y2 = _lrelu(_combine_taps(b) + b2)

        o_ref[img] = ((y2 + ident) * INV_SQRT2).reshape(PLANES, H, W)


def _tap_major(w):
    cout, cin = w.shape[0], w.shape[1]
    return w.transpose(2, 3, 0, 1).reshape(TAPS * cout, cin)


def kernel(w1, b1, w2, b2, wproj, bproj, x):
    B, Cin = x.shape[0], x.shape[1]
    x8 = x.astype(jnp.float32).reshape(B, Cin, 8, 128)

    w1t = _tap_major(w1.astype(jnp.float32)).astype(jnp.bfloat16)
    w2t = _tap_major(w2.astype(jnp.float32)).astype(jnp.bfloat16)
    wp = wproj.astype(jnp.bfloat16).reshape(PLANES, Cin)

    bias = jnp.stack([b1, b2, bproj], axis=1).astype(jnp.float32)

    span = jnp.arange(H, dtype=jnp.float32) / (H - 1) * 2.0 - 1.0
    xx = jnp.broadcast_to(span[:, None], (H, W)).reshape(1, HW)
    yy = jnp.broadcast_to(span[None, :], (H, W)).reshape(1, HW)
    coords = jnp.concatenate([xx, yy], axis=0).astype(jnp.bfloat16)

    out = pl.pallas_call(
        _block_kernel,
        grid=(B // IMGS_PER_STEP,),
        out_shape=jax.ShapeDtypeStruct((B, PLANES, H, W), jnp.float32),
        in_specs=[
            pl.BlockSpec((IMGS_PER_STEP, Cin, 8, 128), lambda i: (i, 0, 0, 0)),
            pl.BlockSpec(w1t.shape, lambda i: (0, 0)),
            pl.BlockSpec(wp.shape, lambda i: (0, 0)),
            pl.BlockSpec(w2t.shape, lambda i: (0, 0)),
            pl.BlockSpec(bias.shape, lambda i: (0, 0)),
            pl.BlockSpec(coords.shape, lambda i: (0, 0)),
        ],
        out_specs=pl.BlockSpec((IMGS_PER_STEP, PLANES, H, W),
                               lambda i: (i, 0, 0, 0)),
        compiler_params=pltpu.CompilerParams(
            dimension_semantics=("parallel",)),
    )(x8, w1t, wp, w2t, bias, coords)

    return out
